# Initial kernel scaffold; baseline (speedup 1.0000x reference)
import jax
import jax.numpy as jnp
from jax.experimental import pallas as pl


def kernel(x, edge_index, batch, W_l1, b_l1, W_r1, W_l2, b_l2, W_r2, W_out, b_out):
    raise NotImplementedError("write your pallas kernel here")


# speedup vs baseline: 11.1187x; 11.1187x over previous
import functools

import jax
import jax.numpy as jnp
from jax import lax
from jax.experimental import pallas as pl
from jax.experimental.pallas import tpu as pltpu
from jax.experimental.pallas import tpu_sc as plsc

N = 10000
E = 320000
DIN = 128
H = 64
G = 64

NC = 2
NS = 16
CW = 128
CPT = 80
EPAD = NC * NS * CPT * CW
NPAD = N + 8
RB = N // 10

_HIGH = jax.lax.Precision.HIGHEST


def _dot(a, b):
    return jax.lax.dot_general(a, b, (((1,), (0,)), ((), ())),
                               precision=_HIGH,
                               preferred_element_type=jnp.float32)


def _make_sc_agg(with_cnt):
    mesh = plsc.VectorSubcoreMesh(core_axis_name="c", subcore_axis_name="s")
    out_type = [jax.ShapeDtypeStruct((N, H), jnp.float32),
                jax.ShapeDtypeStruct((N, H), jnp.float32)]
    scratch = [
        pltpu.VMEM_SHARED((N, H), jnp.float32),
        pltpu.VMEM_SHARED((NPAD, H), jnp.float32),
        pltpu.VMEM((CPT, CW), jnp.int32),
        pltpu.VMEM((CPT, CW), jnp.int32),
        pltpu.VMEM((CW, H), jnp.float32),
    ]
    if with_cnt:
        out_type += [jax.ShapeDtypeStruct((N,), jnp.float32),
                     jax.ShapeDtypeStruct((N,), jnp.float32)]
        scratch += [
            pltpu.VMEM_SHARED((NPAD,), jnp.float32),
            pltpu.VMEM((128,), jnp.float32),
            pltpu.VMEM((RB + 8, ), jnp.float32),
        ]

    def body(*refs):
        if with_cnt:
            (y_hbm, src_hbm, dst_hbm, za_hbm,
             p0_hbm, p1_hbm, c0_hbm, c1_hbm,
             y_s, agg_s, src_v, dst_v, rows_v, cnt_s, ones_v, zb_v) = refs
        else:
            (y_hbm, src_hbm, dst_hbm, za_hbm,
             p0_hbm, p1_hbm,
             y_s, agg_s, src_v, dst_v, rows_v) = refs
        cid = lax.axis_index("c")
        sid = lax.axis_index("s")
        r0 = sid * RB

        if with_cnt:
            @pl.loop(0, 128, step=16)
            def _(i):
                ones_v[pl.ds(i, 16)] = jnp.full((16,), 1.0, jnp.float32)

            @pl.loop(0, RB + 8, step=16)
            def _(i):
                zb_v[pl.ds(i, 16)] = jnp.zeros((16,), jnp.float32)

        @pl.when(sid < 10)
        def _():
            pltpu.sync_copy(y_hbm.at[pl.ds(r0, RB)], y_s.at[pl.ds(r0, RB)])
            pltpu.sync_copy(za_hbm.at[pl.ds(r0, RB)], agg_s.at[pl.ds(r0, RB)])
            if with_cnt:
                pltpu.sync_copy(zb_v.at[pl.ds(0, RB)],
                                cnt_s.at[pl.ds(r0, RB)])

        row0 = (cid * NS + sid) * CPT
        pltpu.sync_copy(src_hbm.at[pl.ds(row0, CPT)], src_v)
        pltpu.sync_copy(dst_hbm.at[pl.ds(row0, CPT)], dst_v)
        plsc.subcore_barrier()

        @pl.loop(0, CPT)
        def _(j):
            pltpu.sync_copy(y_s.at[src_v.at[j]], rows_v)
            pltpu.sync_copy(rows_v, agg_s.at[dst_v.at[j]], add=True)
            if with_cnt:
                pltpu.sync_copy(ones_v.at[pl.ds(0, CW)],
                                cnt_s.at[dst_v.at[j]], add=True)

        plsc.subcore_barrier()

        @pl.when(jnp.logical_and(sid < 10, cid == 0))
        def _():
            pltpu.sync_copy(agg_s.at[pl.ds(r0, RB)], p0_hbm.at[pl.ds(r0, RB)])
            if with_cnt:
                pltpu.sync_copy(cnt_s.at[pl.ds(r0, RB)], zb_v.at[pl.ds(0, RB)])
                pltpu.sync_copy(zb_v.at[pl.ds(0, RB)], c0_hbm.at[pl.ds(r0, RB)])

        @pl.when(jnp.logical_and(sid < 10, cid == 1))
        def _():
            pltpu.sync_copy(agg_s.at[pl.ds(r0, RB)], p1_hbm.at[pl.ds(r0, RB)])
            if with_cnt:
                pltpu.sync_copy(cnt_s.at[pl.ds(r0, RB)], zb_v.at[pl.ds(0, RB)])
                pltpu.sync_copy(zb_v.at[pl.ds(0, RB)], c1_hbm.at[pl.ds(r0, RB)])

    return pl.kernel(body, out_type=out_type, mesh=mesh,
                     compiler_params=pltpu.CompilerParams(
                         use_tc_tiling_on_sc=False),
                     scratch_types=scratch)


_sc_agg_cnt = _make_sc_agg(True)
_sc_agg = _make_sc_agg(False)


_NB = 10
_BN = N // _NB


def _tc_transform(x, wcat):
    def body(x_ref, w_ref, y_ref, r_ref):
        res = _dot(x_ref[...], w_ref[...])
        y_ref[...] = res[:, :H]
        r_ref[...] = res[:, H:]

    return pl.pallas_call(
        body,
        grid=(_NB,),
        in_specs=[pl.BlockSpec((_BN, DIN), lambda i: (i, 0)),
                  pl.BlockSpec((DIN, 2 * H), lambda i: (0, 0))],
        out_specs=[pl.BlockSpec((_BN, H), lambda i: (i, 0)),
                   pl.BlockSpec((_BN, H), lambda i: (i, 0))],
        out_shape=[jax.ShapeDtypeStruct((N, H), jnp.float32),
                   jax.ShapeDtypeStruct((N, H), jnp.float32)],
    )(x, wcat)


def _tc_mid(p0, p1, c0, c1, r1, b1, wcat2):
    def body(p0r, p1r, c0r, c1r, r1r, b1r, wr, y2r, r2r):
        cnt = jnp.maximum(c0r[0, 0] + c1r[0, 0], 1.0)
        inv = (1.0 / cnt)[:, None]
        h = jnp.maximum((p0r[...] + p1r[...]) * inv + b1r[...] + r1r[...],
                        0.0)
        res = _dot(h, wr[...])
        y2r[...] = res[:, :H]
        r2r[...] = res[:, H:]

    return pl.pallas_call(
        body,
        grid=(_NB,),
        in_specs=[pl.BlockSpec((_BN, H), lambda i: (i, 0)),
                  pl.BlockSpec((_BN, H), lambda i: (i, 0)),
                  pl.BlockSpec((1, 1, _BN), lambda i: (i, 0, 0)),
                  pl.BlockSpec((1, 1, _BN), lambda i: (i, 0, 0)),
                  pl.BlockSpec((_BN, H), lambda i: (i, 0)),
                  pl.BlockSpec((1, H), lambda i: (0, 0)),
                  pl.BlockSpec((H, 2 * H), lambda i: (0, 0))],
        out_specs=[pl.BlockSpec((_BN, H), lambda i: (i, 0)),
                   pl.BlockSpec((_BN, H), lambda i: (i, 0))],
        out_shape=[jax.ShapeDtypeStruct((N, H), jnp.float32),
                   jax.ShapeDtypeStruct((N, H), jnp.float32)],
    )(p0, p1, c0, c1, r1, b1, wcat2)


def _tc_final(q0, q1, c0, c1, r2, b2, batch3, wrow, brow):
    def body(q0r, q1r, c0r, c1r, r2r, b2r, br, wror, bror, out_ref, acc):
        i = pl.program_id(0)

        @pl.when(i == 0)
        def _():
            acc[...] = jnp.zeros_like(acc)

        cnt = jnp.maximum(c0r[0, 0] + c1r[0, 0], 1.0)
        inv = (1.0 / cnt)[:, None]
        h = jnp.maximum((q0r[...] + q1r[...]) * inv + b2r[...] + r2r[...],
                        0.0)
        hcat = jnp.concatenate(
            [h, jnp.ones((_BN, 1), jnp.float32),
             jnp.zeros((_BN, DIN - H - 1), jnp.float32)], axis=1)
        b = br[0, 0]
        mask = (lax.broadcasted_iota(jnp.int32, (G, _BN), 0)
                == b[None, :]).astype(jnp.float32)
        acc[...] += _dot(mask, hcat)

        @pl.when(i == _NB - 1)
        def _():
            pooled = acc[:, :H] / jnp.maximum(acc[:, H:H + 1], 1.0)
            out_ref[...] = jnp.sum(pooled * wror[...], axis=1) + bror[0]

    return pl.pallas_call(
        body,
        grid=(_NB,),
        in_specs=[pl.BlockSpec((_BN, H), lambda i: (i, 0)),
                  pl.BlockSpec((_BN, H), lambda i: (i, 0)),
                  pl.BlockSpec((1, 1, _BN), lambda i: (i, 0, 0)),
                  pl.BlockSpec((1, 1, _BN), lambda i: (i, 0, 0)),
                  pl.BlockSpec((_BN, H), lambda i: (i, 0)),
                  pl.BlockSpec((1, H), lambda i: (0, 0)),
                  pl.BlockSpec((1, 1, _BN), lambda i: (i, 0, 0)),
                  pl.BlockSpec((1, H), lambda i: (0, 0)),
                  pl.BlockSpec((1, H), lambda i: (0, 0))],
        out_specs=pl.BlockSpec((G,), lambda i: (0,)),
        out_shape=jax.ShapeDtypeStruct((G,), jnp.float32),
        scratch_shapes=[pltpu.VMEM((G, DIN), jnp.float32)],
    )(q0, q1, c0, c1, r2, b2, batch3, wrow, brow)


def kernel(x, edge_index, batch, W_l1, b_l1, W_r1, W_l2, b_l2, W_r2,
           W_out, b_out):
    wcat1 = jnp.concatenate([W_l1, W_r1], axis=1)
    wcat2 = jnp.concatenate([W_l2, W_r2], axis=1)
    npad = EPAD - E
    src2 = jnp.concatenate(
        [edge_index[0], jnp.zeros((npad,), jnp.int32)]).reshape(EPAD // CW, CW)
    dst2 = jnp.concatenate(
        [edge_index[1], jnp.full((npad,), N, jnp.int32)]).reshape(EPAD // CW,
                                                                  CW)
    za = jnp.zeros((N, H), jnp.float32)
    b1 = b_l1.reshape(1, H)
    b2 = b_l2.reshape(1, H)
    batch3 = batch.reshape(_NB, 1, _BN)
    wrow = W_out.reshape(1, H)
    brow = jnp.broadcast_to(b_out.reshape(1, 1), (1, H))

    y1, r1 = _tc_transform(x, wcat1)
    p0, p1, c0, c1 = _sc_agg_cnt(y1, src2, dst2, za)
    c03 = c0.reshape(_NB, 1, _BN)
    c13 = c1.reshape(_NB, 1, _BN)
    y2, r2 = _tc_mid(p0, p1, c03, c13, r1, b1, wcat2)
    q0, q1 = _sc_agg(y2, src2, dst2, za)
    return _tc_final(q0, q1, c03, c13, r2, b2, batch3, wrow, brow)


# speedup vs baseline: 13.0504x; 1.1737x over previous
import functools

import jax
import jax.numpy as jnp
from jax import lax
from jax.experimental import pallas as pl
from jax.experimental.pallas import tpu as pltpu
from jax.experimental.pallas import tpu_sc as plsc

N = 10000
E = 320000
DIN = 128
H = 64
G = 64

NC = 2
NS = 16
CW = 128
CPT = 80
EPAD = NC * NS * CPT * CW
NPAD = N + 8
RB = N // 10
NBUF = 2

_HIGH = jax.lax.Precision.HIGHEST


def _dot(a, b):
    return jax.lax.dot_general(a, b, (((1,), (0,)), ((), ())),
                               precision=_HIGH,
                               preferred_element_type=jnp.float32)


def _make_sc_agg(with_cnt):
    mesh = plsc.VectorSubcoreMesh(core_axis_name="c", subcore_axis_name="s")
    out_type = [jax.ShapeDtypeStruct((N, H), jnp.float32),
                jax.ShapeDtypeStruct((N, H), jnp.float32)]
    scratch = [
        pltpu.VMEM_SHARED((N, H), jnp.float32),
        pltpu.VMEM_SHARED((NPAD, H), jnp.float32),
        pltpu.VMEM((CPT, CW), jnp.int32),
        pltpu.VMEM((CPT, CW), jnp.int32),
        [pltpu.VMEM((CW, H), jnp.float32)] * NBUF,
        [pltpu.SemaphoreType.DMA] * NBUF,
        [pltpu.SemaphoreType.DMA] * NBUF,
        pltpu.SemaphoreType.DMA,
    ]
    if with_cnt:
        out_type += [jax.ShapeDtypeStruct((N,), jnp.float32),
                     jax.ShapeDtypeStruct((N,), jnp.float32)]
        scratch += [
            pltpu.VMEM_SHARED((NPAD,), jnp.float32),
            pltpu.VMEM((128,), jnp.float32),
            pltpu.VMEM((RB + 8, ), jnp.float32),
            pltpu.SemaphoreType.DMA,
        ]

    def body(*refs):
        if with_cnt:
            (y_hbm, src_hbm, dst_hbm, za_hbm,
             p0_hbm, p1_hbm, c0_hbm, c1_hbm,
             y_s, agg_s, src_v, dst_v, rows, gsem, ssem, isem,
             cnt_s, ones_v, zb_v, csem) = refs
        else:
            (y_hbm, src_hbm, dst_hbm, za_hbm,
             p0_hbm, p1_hbm,
             y_s, agg_s, src_v, dst_v, rows, gsem, ssem, isem) = refs
        cid = lax.axis_index("c")
        sid = lax.axis_index("s")
        r0 = sid * RB
        rr0 = sid * (N // NS)

        t_cp = pltpu.async_copy(y_hbm.at[pl.ds(rr0, N // NS)],
                                y_s.at[pl.ds(rr0, N // NS)], isem)
        z_cp = pltpu.async_copy(za_hbm.at[pl.ds(rr0, N // NS)],
                                agg_s.at[pl.ds(rr0, N // NS)], isem)
        row0 = (cid * NS + sid) * CPT
        pltpu.sync_copy(src_hbm.at[pl.ds(row0, CPT)], src_v)
        pltpu.sync_copy(dst_hbm.at[pl.ds(row0, CPT)], dst_v)

        if with_cnt:
            @pl.loop(0, 128, step=16)
            def _(i):
                ones_v[pl.ds(i, 16)] = jnp.full((16,), 1.0, jnp.float32)

            @pl.loop(0, RB + 8, step=16)
            def _(i):
                zb_v[pl.ds(i, 16)] = jnp.zeros((16,), jnp.float32)

            @pl.when(sid < 10)
            def _():
                pltpu.sync_copy(zb_v.at[pl.ds(0, RB)],
                                cnt_s.at[pl.ds(r0, RB)])
        t_cp.wait()
        z_cp.wait()
        plsc.subcore_barrier()

        def g_cp(j, b):
            return pltpu.make_async_copy(y_s.at[src_v.at[j]], rows[b],
                                         gsem[b])

        def s_cp(j, b):
            return pltpu.make_async_copy(rows[b], agg_s.at[dst_v.at[j]],
                                         ssem[b])

        for b in range(NBUF):
            pltpu.async_copy(y_s.at[src_v.at[b]], rows[b], gsem[b])

        @pl.loop(0, CPT, step=NBUF)
        def _(j):
            for b in range(NBUF):
                g_cp(j + b, b).wait()
                pltpu.async_copy(rows[b], agg_s.at[dst_v.at[j + b]],
                                 ssem[b], add=True)
                if with_cnt:
                    pltpu.async_copy(ones_v.at[pl.ds(0, CW)],
                                     cnt_s.at[dst_v.at[j + b]], csem,
                                     add=True)
            for b in range(NBUF):
                s_cp(j + b, b).wait()

                @pl.when(j + b + NBUF < CPT)
                def _():
                    pltpu.async_copy(y_s.at[src_v.at[j + b + NBUF]], rows[b],
                                     gsem[b])
            if with_cnt:
                @pl.loop(0, NBUF)
                def _(_k):
                    pltpu.make_async_copy(ones_v.at[pl.ds(0, CW)],
                                          cnt_s.at[dst_v.at[j]],
                                          csem).wait()

        plsc.subcore_barrier()

        @pl.when(cid == 0)
        def _():
            pltpu.sync_copy(agg_s.at[pl.ds(rr0, N // NS)],
                            p0_hbm.at[pl.ds(rr0, N // NS)])

        @pl.when(cid == 1)
        def _():
            pltpu.sync_copy(agg_s.at[pl.ds(rr0, N // NS)],
                            p1_hbm.at[pl.ds(rr0, N // NS)])

        if with_cnt:
            @pl.when(jnp.logical_and(sid < 10, cid == 0))
            def _():
                pltpu.sync_copy(cnt_s.at[pl.ds(r0, RB)], zb_v.at[pl.ds(0, RB)])
                pltpu.sync_copy(zb_v.at[pl.ds(0, RB)], c0_hbm.at[pl.ds(r0, RB)])

            @pl.when(jnp.logical_and(sid < 10, cid == 1))
            def _():
                pltpu.sync_copy(cnt_s.at[pl.ds(r0, RB)], zb_v.at[pl.ds(0, RB)])
                pltpu.sync_copy(zb_v.at[pl.ds(0, RB)], c1_hbm.at[pl.ds(r0, RB)])

    return pl.kernel(body, out_type=out_type, mesh=mesh,
                     compiler_params=pltpu.CompilerParams(
                         use_tc_tiling_on_sc=False),
                     scratch_types=scratch)


_sc_agg_cnt = _make_sc_agg(True)
_sc_agg = _make_sc_agg(False)


_NB = 10
_BN = N // _NB


def _tc_transform(x, wcat):
    def body(x_ref, w_ref, y_ref, r_ref):
        res = _dot(x_ref[...], w_ref[...])
        y_ref[...] = res[:, :H]
        r_ref[...] = res[:, H:]

    return pl.pallas_call(
        body,
        grid=(_NB,),
        in_specs=[pl.BlockSpec((_BN, DIN), lambda i: (i, 0)),
                  pl.BlockSpec((DIN, 2 * H), lambda i: (0, 0))],
        out_specs=[pl.BlockSpec((_BN, H), lambda i: (i, 0)),
                   pl.BlockSpec((_BN, H), lambda i: (i, 0))],
        out_shape=[jax.ShapeDtypeStruct((N, H), jnp.float32),
                   jax.ShapeDtypeStruct((N, H), jnp.float32)],
    )(x, wcat)


def _tc_mid(p0, p1, c0, c1, r1, b1, wcat2):
    def body(p0r, p1r, c0r, c1r, r1r, b1r, wr, y2r, r2r):
        cnt = jnp.maximum(c0r[0, 0] + c1r[0, 0], 1.0)
        inv = (1.0 / cnt)[:, None]
        h = jnp.maximum((p0r[...] + p1r[...]) * inv + b1r[...] + r1r[...],
                        0.0)
        res = _dot(h, wr[...])
        y2r[...] = res[:, :H]
        r2r[...] = res[:, H:]

    return pl.pallas_call(
        body,
        grid=(_NB,),
        in_specs=[pl.BlockSpec((_BN, H), lambda i: (i, 0)),
                  pl.BlockSpec((_BN, H), lambda i: (i, 0)),
                  pl.BlockSpec((1, 1, _BN), lambda i: (i, 0, 0)),
                  pl.BlockSpec((1, 1, _BN), lambda i: (i, 0, 0)),
                  pl.BlockSpec((_BN, H), lambda i: (i, 0)),
                  pl.BlockSpec((1, H), lambda i: (0, 0)),
                  pl.BlockSpec((H, 2 * H), lambda i: (0, 0))],
        out_specs=[pl.BlockSpec((_BN, H), lambda i: (i, 0)),
                   pl.BlockSpec((_BN, H), lambda i: (i, 0))],
        out_shape=[jax.ShapeDtypeStruct((N, H), jnp.float32),
                   jax.ShapeDtypeStruct((N, H), jnp.float32)],
    )(p0, p1, c0, c1, r1, b1, wcat2)


def _tc_final(q0, q1, c0, c1, r2, b2, batch3, wrow, brow):
    def body(q0r, q1r, c0r, c1r, r2r, b2r, br, wror, bror, out_ref, acc):
        i = pl.program_id(0)

        @pl.when(i == 0)
        def _():
            acc[...] = jnp.zeros_like(acc)

        cnt = jnp.maximum(c0r[0, 0] + c1r[0, 0], 1.0)
        inv = (1.0 / cnt)[:, None]
        h = jnp.maximum((q0r[...] + q1r[...]) * inv + b2r[...] + r2r[...],
                        0.0)
        hcat = jnp.concatenate(
            [h, jnp.ones((_BN, 1), jnp.float32),
             jnp.zeros((_BN, DIN - H - 1), jnp.float32)], axis=1)
        b = br[0, 0]
        mask = (lax.broadcasted_iota(jnp.int32, (G, _BN), 0)
                == b[None, :]).astype(jnp.float32)
        acc[...] += _dot(mask, hcat)

        @pl.when(i == _NB - 1)
        def _():
            pooled = acc[:, :H] / jnp.maximum(acc[:, H:H + 1], 1.0)
            out_ref[...] = jnp.sum(pooled * wror[...], axis=1) + bror[0]

    return pl.pallas_call(
        body,
        grid=(_NB,),
        in_specs=[pl.BlockSpec((_BN, H), lambda i: (i, 0)),
                  pl.BlockSpec((_BN, H), lambda i: (i, 0)),
                  pl.BlockSpec((1, 1, _BN), lambda i: (i, 0, 0)),
                  pl.BlockSpec((1, 1, _BN), lambda i: (i, 0, 0)),
                  pl.BlockSpec((_BN, H), lambda i: (i, 0)),
                  pl.BlockSpec((1, H), lambda i: (0, 0)),
                  pl.BlockSpec((1, 1, _BN), lambda i: (i, 0, 0)),
                  pl.BlockSpec((1, H), lambda i: (0, 0)),
                  pl.BlockSpec((1, H), lambda i: (0, 0))],
        out_specs=pl.BlockSpec((G,), lambda i: (0,)),
        out_shape=jax.ShapeDtypeStruct((G,), jnp.float32),
        scratch_shapes=[pltpu.VMEM((G, DIN), jnp.float32)],
    )(q0, q1, c0, c1, r2, b2, batch3, wrow, brow)


def kernel(x, edge_index, batch, W_l1, b_l1, W_r1, W_l2, b_l2, W_r2,
           W_out, b_out):
    wcat1 = jnp.concatenate([W_l1, W_r1], axis=1)
    wcat2 = jnp.concatenate([W_l2, W_r2], axis=1)
    npad = EPAD - E
    src2 = jnp.concatenate(
        [edge_index[0], jnp.zeros((npad,), jnp.int32)]).reshape(EPAD // CW, CW)
    dst2 = jnp.concatenate(
        [edge_index[1], jnp.full((npad,), N, jnp.int32)]).reshape(EPAD // CW,
                                                                  CW)
    za = jnp.zeros((N, H), jnp.float32)
    b1 = b_l1.reshape(1, H)
    b2 = b_l2.reshape(1, H)
    batch3 = batch.reshape(_NB, 1, _BN)
    wrow = W_out.reshape(1, H)
    brow = jnp.broadcast_to(b_out.reshape(1, 1), (1, H))

    y1, r1 = _tc_transform(x, wcat1)
    p0, p1, c0, c1 = _sc_agg_cnt(y1, src2, dst2, za)
    c03 = c0.reshape(_NB, 1, _BN)
    c13 = c1.reshape(_NB, 1, _BN)
    y2, r2 = _tc_mid(p0, p1, c03, c13, r1, b1, wcat2)
    q0, q1 = _sc_agg(y2, src2, dst2, za)
    return _tc_final(q0, q1, c03, c13, r2, b2, batch3, wrow, brow)


# speedup vs baseline: 13.1297x; 1.0061x over previous
import functools

import jax
import jax.numpy as jnp
from jax import lax
from jax.experimental import pallas as pl
from jax.experimental.pallas import tpu as pltpu
from jax.experimental.pallas import tpu_sc as plsc

N = 10000
E = 320000
DIN = 128
H = 64
G = 64

NC = 2
NS = 16
CW = 128
CPT = 80
EPAD = NC * NS * CPT * CW
NPAD = N + 8
RB = N // 10
NBUF = 2

_HIGH = jax.lax.Precision.HIGHEST


def _dot(a, b):
    return jax.lax.dot_general(a, b, (((1,), (0,)), ((), ())),
                               precision=_HIGH,
                               preferred_element_type=jnp.float32)


def _make_sc_agg(with_cnt):
    mesh = plsc.VectorSubcoreMesh(core_axis_name="c", subcore_axis_name="s")
    out_type = [jax.ShapeDtypeStruct((N, H), jnp.float32),
                jax.ShapeDtypeStruct((N, H), jnp.float32)]
    scratch = [
        pltpu.VMEM_SHARED((N, H), jnp.float32),
        pltpu.VMEM_SHARED((NPAD, H), jnp.float32),
        pltpu.VMEM((CPT, CW), jnp.int32),
        pltpu.VMEM((CPT, CW), jnp.int32),
        [pltpu.VMEM((CW, H), jnp.float32)] * NBUF,
        [pltpu.SemaphoreType.DMA] * NBUF,
        [pltpu.SemaphoreType.DMA] * NBUF,
        pltpu.SemaphoreType.DMA,
    ]
    if with_cnt:
        out_type += [jax.ShapeDtypeStruct((N // RB, 1, RB), jnp.float32),
                     jax.ShapeDtypeStruct((N // RB, 1, RB), jnp.float32)]
        scratch += [
            pltpu.VMEM_SHARED((NPAD,), jnp.float32),
            pltpu.VMEM((128,), jnp.float32),
            pltpu.VMEM((RB + 8, ), jnp.float32),
            pltpu.SemaphoreType.DMA,
        ]

    def body(*refs):
        if with_cnt:
            (y_hbm, src_hbm, dst_hbm, za_hbm,
             p0_hbm, p1_hbm, c0_hbm, c1_hbm,
             y_s, agg_s, src_v, dst_v, rows, gsem, ssem, isem,
             cnt_s, ones_v, zb_v, csem) = refs
        else:
            (y_hbm, src_hbm, dst_hbm, za_hbm,
             p0_hbm, p1_hbm,
             y_s, agg_s, src_v, dst_v, rows, gsem, ssem, isem) = refs
        cid = lax.axis_index("c")
        sid = lax.axis_index("s")
        r0 = sid * RB
        rr0 = sid * (N // NS)

        t_cp = pltpu.async_copy(y_hbm.at[pl.ds(rr0, N // NS)],
                                y_s.at[pl.ds(rr0, N // NS)], isem)
        z_cp = pltpu.async_copy(za_hbm.at[pl.ds(rr0, N // NS)],
                                agg_s.at[pl.ds(rr0, N // NS)], isem)
        row0 = (cid * NS + sid) * CPT
        pltpu.sync_copy(src_hbm.at[pl.ds(row0, CPT)], src_v)
        pltpu.sync_copy(dst_hbm.at[pl.ds(row0, CPT)], dst_v)

        if with_cnt:
            @pl.loop(0, 128, step=16)
            def _(i):
                ones_v[pl.ds(i, 16)] = jnp.full((16,), 1.0, jnp.float32)

            @pl.loop(0, RB + 8, step=16)
            def _(i):
                zb_v[pl.ds(i, 16)] = jnp.zeros((16,), jnp.float32)

            @pl.when(sid < 10)
            def _():
                pltpu.sync_copy(zb_v.at[pl.ds(0, RB)],
                                cnt_s.at[pl.ds(r0, RB)])
        t_cp.wait()
        z_cp.wait()
        plsc.subcore_barrier()

        def g_cp(j, b):
            return pltpu.make_async_copy(y_s.at[src_v.at[j]], rows[b],
                                         gsem[b])

        def s_cp(j, b):
            return pltpu.make_async_copy(rows[b], agg_s.at[dst_v.at[j]],
                                         ssem[b])

        for b in range(NBUF):
            pltpu.async_copy(y_s.at[src_v.at[b]], rows[b], gsem[b])

        @pl.loop(0, CPT, step=NBUF)
        def _(j):
            for b in range(NBUF):
                g_cp(j + b, b).wait()
                pltpu.async_copy(rows[b], agg_s.at[dst_v.at[j + b]],
                                 ssem[b], add=True)
                if with_cnt:
                    pltpu.async_copy(ones_v.at[pl.ds(0, CW)],
                                     cnt_s.at[dst_v.at[j + b]], csem,
                                     add=True)
            for b in range(NBUF):
                s_cp(j + b, b).wait()

                @pl.when(j + b + NBUF < CPT)
                def _():
                    pltpu.async_copy(y_s.at[src_v.at[j + b + NBUF]], rows[b],
                                     gsem[b])
            if with_cnt:
                @pl.loop(0, NBUF)
                def _(_k):
                    pltpu.make_async_copy(ones_v.at[pl.ds(0, CW)],
                                          cnt_s.at[dst_v.at[j]],
                                          csem).wait()

        plsc.subcore_barrier()

        @pl.when(cid == 0)
        def _():
            pltpu.sync_copy(agg_s.at[pl.ds(rr0, N // NS)],
                            p0_hbm.at[pl.ds(rr0, N // NS)])

        @pl.when(cid == 1)
        def _():
            pltpu.sync_copy(agg_s.at[pl.ds(rr0, N // NS)],
                            p1_hbm.at[pl.ds(rr0, N // NS)])

        if with_cnt:
            @pl.when(jnp.logical_and(sid < 10, cid == 0))
            def _():
                pltpu.sync_copy(cnt_s.at[pl.ds(r0, RB)], zb_v.at[pl.ds(0, RB)])
                pltpu.sync_copy(zb_v.at[pl.ds(0, RB)], c0_hbm.at[sid, 0])

            @pl.when(jnp.logical_and(sid < 10, cid == 1))
            def _():
                pltpu.sync_copy(cnt_s.at[pl.ds(r0, RB)], zb_v.at[pl.ds(0, RB)])
                pltpu.sync_copy(zb_v.at[pl.ds(0, RB)], c1_hbm.at[sid, 0])

    return pl.kernel(body, out_type=out_type, mesh=mesh,
                     compiler_params=pltpu.CompilerParams(
                         use_tc_tiling_on_sc=False),
                     scratch_types=scratch)


_sc_agg_cnt = _make_sc_agg(True)
_sc_agg = _make_sc_agg(False)


_NB = 10
_BN = N // _NB


def _tc_matmul(x, w):
    k = x.shape[1]
    m = w.shape[1]

    def body(x_ref, w_ref, y_ref):
        y_ref[...] = _dot(x_ref[...], w_ref[...])

    return pl.pallas_call(
        body,
        grid=(_NB,),
        in_specs=[pl.BlockSpec((_BN, k), lambda i: (i, 0)),
                  pl.BlockSpec((k, m), lambda i: (0, 0))],
        out_specs=pl.BlockSpec((_BN, m), lambda i: (i, 0)),
        out_shape=jax.ShapeDtypeStruct((N, m), jnp.float32),
    )(x, w)


def _tc_mid(p0, p1, c0, c1, r1, b1, wl2):
    def body(p0r, p1r, c0r, c1r, r1r, b1r, wr, y2r, hr):
        cnt = jnp.maximum(c0r[0, 0] + c1r[0, 0], 1.0)
        inv = (1.0 / cnt)[:, None]
        h = jnp.maximum((p0r[...] + p1r[...]) * inv + b1r[...] + r1r[...],
                        0.0)
        y2r[...] = _dot(h, wr[...])
        hr[...] = h

    return pl.pallas_call(
        body,
        grid=(_NB,),
        in_specs=[pl.BlockSpec((_BN, H), lambda i: (i, 0)),
                  pl.BlockSpec((_BN, H), lambda i: (i, 0)),
                  pl.BlockSpec((1, 1, _BN), lambda i: (i, 0, 0)),
                  pl.BlockSpec((1, 1, _BN), lambda i: (i, 0, 0)),
                  pl.BlockSpec((_BN, H), lambda i: (i, 0)),
                  pl.BlockSpec((1, H), lambda i: (0, 0)),
                  pl.BlockSpec((H, H), lambda i: (0, 0))],
        out_specs=[pl.BlockSpec((_BN, H), lambda i: (i, 0)),
                   pl.BlockSpec((_BN, H), lambda i: (i, 0))],
        out_shape=[jax.ShapeDtypeStruct((N, H), jnp.float32),
                   jax.ShapeDtypeStruct((N, H), jnp.float32)],
    )(p0, p1, c0, c1, r1, b1, wl2)


def _tc_final(q0, q1, c0, c1, r2, b2, batch3, wrow, brow):
    def body(q0r, q1r, c0r, c1r, r2r, b2r, br, wror, bror, out_ref, acc):
        i = pl.program_id(0)

        @pl.when(i == 0)
        def _():
            acc[...] = jnp.zeros_like(acc)

        cnt = jnp.maximum(c0r[0, 0] + c1r[0, 0], 1.0)
        inv = (1.0 / cnt)[:, None]
        h = jnp.maximum((q0r[...] + q1r[...]) * inv + b2r[...] + r2r[...],
                        0.0)
        hcat = jnp.concatenate(
            [h, jnp.ones((_BN, 1), jnp.float32),
             jnp.zeros((_BN, DIN - H - 1), jnp.float32)], axis=1)
        b = br[0, 0]
        mask = (lax.broadcasted_iota(jnp.int32, (G, _BN), 0)
                == b[None, :]).astype(jnp.float32)
        acc[...] += _dot(mask, hcat)

        @pl.when(i == _NB - 1)
        def _():
            pooled = acc[:, :H] / jnp.maximum(acc[:, H:H + 1], 1.0)
            out_ref[...] = jnp.sum(pooled * wror[...], axis=1) + bror[0]

    return pl.pallas_call(
        body,
        grid=(_NB,),
        in_specs=[pl.BlockSpec((_BN, H), lambda i: (i, 0)),
                  pl.BlockSpec((_BN, H), lambda i: (i, 0)),
                  pl.BlockSpec((1, 1, _BN), lambda i: (i, 0, 0)),
                  pl.BlockSpec((1, 1, _BN), lambda i: (i, 0, 0)),
                  pl.BlockSpec((_BN, H), lambda i: (i, 0)),
                  pl.BlockSpec((1, H), lambda i: (0, 0)),
                  pl.BlockSpec((1, 1, _BN), lambda i: (i, 0, 0)),
                  pl.BlockSpec((1, H), lambda i: (0, 0)),
                  pl.BlockSpec((1, H), lambda i: (0, 0))],
        out_specs=pl.BlockSpec((G,), lambda i: (0,)),
        out_shape=jax.ShapeDtypeStruct((G,), jnp.float32),
        scratch_shapes=[pltpu.VMEM((G, DIN), jnp.float32)],
    )(q0, q1, c0, c1, r2, b2, batch3, wrow, brow)


def kernel(x, edge_index, batch, W_l1, b_l1, W_r1, W_l2, b_l2, W_r2,
           W_out, b_out):
    npad = EPAD - E
    src2 = jnp.concatenate(
        [edge_index[0], jnp.zeros((npad,), jnp.int32)]).reshape(EPAD // CW, CW)
    dst2 = jnp.concatenate(
        [edge_index[1], jnp.full((npad,), N, jnp.int32)]).reshape(EPAD // CW,
                                                                  CW)
    za = jnp.zeros((N, H), jnp.float32)
    b1 = b_l1.reshape(1, H)
    b2 = b_l2.reshape(1, H)
    batch3 = batch.reshape(_NB, 1, _BN)
    wrow = W_out.reshape(1, H)
    brow = jnp.broadcast_to(b_out.reshape(1, 1), (1, H))

    y1 = _tc_matmul(x, W_l1)
    p0, p1, c03, c13 = _sc_agg_cnt(y1, src2, dst2, za)
    r1 = _tc_matmul(x, W_r1)
    y2, h = _tc_mid(p0, p1, c03, c13, r1, b1, W_l2)
    q0, q1 = _sc_agg(y2, src2, dst2, za)
    r2 = _tc_matmul(h, W_r2)
    return _tc_final(q0, q1, c03, c13, r2, b2, batch3, wrow, brow)
